# Initial kernel scaffold; baseline (speedup 1.0000x reference)
import jax
import jax.numpy as jnp
from jax.experimental import pallas as pl


def kernel(idx, table):
    raise NotImplementedError("write your pallas kernel here")


# speedup vs baseline: 1.3551x; 1.3551x over previous
import functools

import jax
import jax.numpy as jnp
from jax import lax
from jax.experimental import pallas as pl
from jax.experimental.pallas import tpu as pltpu
from jax.experimental.pallas import tpu_sc as plsc

VOCAB = 1000
BATCH, TIME = 1024, 20
NROWS = BATCH * TIME
NUM_CORES = 2
NUM_SUBCORES = 16
NW = NUM_CORES * NUM_SUBCORES
ROWS_PER_W = NROWS // NW
CHUNK = 64
NCHUNK = ROWS_PER_W // CHUNK

_MESH = plsc.VectorSubcoreMesh(core_axis_name="c", subcore_axis_name="s")


@functools.partial(
    pl.kernel,
    mesh=_MESH,
    out_type=jax.ShapeDtypeStruct((NROWS, VOCAB), jnp.float32),
    scratch_types=[
        pltpu.VMEM((NCHUNK, CHUNK), jnp.int32),
        pltpu.VMEM((CHUNK, VOCAB), jnp.float32),
        pltpu.VMEM((CHUNK, VOCAB), jnp.float32),
        pltpu.SemaphoreType.DMA,
        pltpu.SemaphoreType.DMA,
    ],
    compiler_params=pltpu.CompilerParams(use_tc_tiling_on_sc=False),
)
def _gather_rows(idx_hbm, table_hbm, out_hbm, idx_v, buf0, buf1, sem0, sem1):
    wid = lax.axis_index("s") * NUM_CORES + lax.axis_index("c")
    base = wid * ROWS_PER_W
    pltpu.sync_copy(idx_hbm.at[wid], idx_v)
    bufs = (buf0, buf1)
    sems = (sem0, sem1)
    handles = [None] * NCHUNK
    handles[0] = pltpu.async_copy(table_hbm.at[idx_v.at[0]], bufs[0], sems[0])
    for c in range(NCHUNK):
        if c + 1 < NCHUNK:
            handles[c + 1] = pltpu.async_copy(
                table_hbm.at[idx_v.at[c + 1]], bufs[(c + 1) % 2], sems[(c + 1) % 2]
            )
        handles[c].wait()
        pltpu.sync_copy(
            bufs[c % 2], out_hbm.at[pl.ds(base + c * CHUNK, CHUNK)]
        )


def kernel(idx, table):
    idx3 = idx.reshape(NW, NCHUNK, CHUNK).astype(jnp.int32)
    out = _gather_rows(idx3, table)
    return out.reshape(BATCH, TIME, VOCAB)


# speedup vs baseline: 1.3566x; 1.0011x over previous
import functools

import jax
import jax.numpy as jnp
from jax import lax
from jax.experimental import pallas as pl
from jax.experimental.pallas import tpu as pltpu
from jax.experimental.pallas import tpu_sc as plsc

VOCAB = 1000
BATCH, TIME = 1024, 20
NROWS = BATCH * TIME
NUM_CORES = 2
NUM_SUBCORES = 16
NW = NUM_CORES * NUM_SUBCORES
ROWS_PER_W = NROWS // NW
CHUNK = 64
NCHUNK = ROWS_PER_W // CHUNK

_MESH = plsc.VectorSubcoreMesh(core_axis_name="c", subcore_axis_name="s")


@functools.partial(
    pl.kernel,
    mesh=_MESH,
    out_type=jax.ShapeDtypeStruct((NROWS, VOCAB), jnp.float32),
    scratch_types=[
        pltpu.VMEM((NCHUNK, CHUNK), jnp.int32),
        pltpu.VMEM((CHUNK, VOCAB), jnp.float32),
        pltpu.VMEM((CHUNK, VOCAB), jnp.float32),
        pltpu.SemaphoreType.DMA,
        pltpu.SemaphoreType.DMA,
        pltpu.SemaphoreType.DMA,
        pltpu.SemaphoreType.DMA,
    ],
    compiler_params=pltpu.CompilerParams(use_tc_tiling_on_sc=False),
)
def _gather_rows(
    idx_hbm, table_hbm, out_hbm, idx_v, buf0, buf1, gsem0, gsem1, wsem0, wsem1
):
    wid = lax.axis_index("s") * NUM_CORES + lax.axis_index("c")
    base = wid * ROWS_PER_W
    pltpu.sync_copy(idx_hbm.at[wid], idx_v)
    bufs = (buf0, buf1)
    gsems = (gsem0, gsem1)
    wsems = (wsem0, wsem1)
    gh = [None] * NCHUNK
    wh = [None] * NCHUNK
    gh[0] = pltpu.async_copy(table_hbm.at[idx_v.at[0]], bufs[0], gsems[0])
    gh[1] = pltpu.async_copy(table_hbm.at[idx_v.at[1]], bufs[1], gsems[1])
    for c in range(NCHUNK):
        b = c % 2
        gh[c].wait()
        wh[c] = pltpu.async_copy(
            bufs[b], out_hbm.at[pl.ds(base + c * CHUNK, CHUNK)], wsems[b]
        )
        if c + 2 < NCHUNK:
            wh[c].wait()
            gh[c + 2] = pltpu.async_copy(
                table_hbm.at[idx_v.at[c + 2]], bufs[b], gsems[b]
            )
    wh[NCHUNK - 2].wait()
    wh[NCHUNK - 1].wait()


def kernel(idx, table):
    idx3 = idx.reshape(NW, NCHUNK, CHUNK).astype(jnp.int32)
    out = _gather_rows(idx3, table)
    return out.reshape(BATCH, TIME, VOCAB)
